# baseline (speedup 1.0000x reference)
import functools

import jax
import jax.numpy as jnp
from jax.experimental import pallas as pl
from jax.experimental.pallas import tpu as pltpu

CUTOFF = 0.5
BLK_I = 256
BLK_J = 1024


def _h_kernel(z_ref, w_ref, b_ref, h_ref):
    h_ref[...] = (
        jnp.dot(z_ref[...], w_ref[...], preferred_element_type=jnp.float32)
        + b_ref[...]
    )


def _agg_kernel(scal_ref, dist_ref, h_ref, out_ref):
    i = pl.program_id(0)
    j = pl.program_id(1)
    inv_mu = scal_ref[0, 0]
    inv_two_sig2 = scal_ref[0, 1]
    d = dist_ref[...]
    delta = 1.0 / d - inv_mu
    sens = jnp.exp(-(delta * delta) * inv_two_sig2)
    rows = jax.lax.broadcasted_iota(jnp.int32, d.shape, 0) + i * BLK_I
    cols = jax.lax.broadcasted_iota(jnp.int32, d.shape, 1) + j * BLK_J
    w = jnp.where((d < CUTOFF) & (rows != cols), sens, 0.0)
    part = jnp.dot(w, h_ref[...], preferred_element_type=jnp.float32)

    @pl.when(j == 0)
    def _init():
        out_ref[...] = part

    @pl.when(j != 0)
    def _acc():
        out_ref[...] += part


@functools.partial(jax.jit, static_argnames=())
def kernel(z, dist_matrix, W, B, mu, sigma):
    n, d_in = z.shape
    d_out = W.shape[1]

    h = pl.pallas_call(
        _h_kernel,
        out_shape=jax.ShapeDtypeStruct((n, d_out), jnp.float32),
    )(z, W, B.reshape(1, d_out))

    scal = jnp.stack([1.0 / mu[0], 1.0 / (2.0 * sigma[0] * sigma[0])]).reshape(1, 2)

    grid = (n // BLK_I, n // BLK_J)
    out = pl.pallas_call(
        _agg_kernel,
        grid=grid,
        in_specs=[
            pl.BlockSpec((1, 2), lambda i, j: (0, 0)),
            pl.BlockSpec((BLK_I, BLK_J), lambda i, j: (i, j)),
            pl.BlockSpec((BLK_J, d_out), lambda i, j: (j, 0)),
        ],
        out_specs=pl.BlockSpec((BLK_I, d_out), lambda i, j: (i, 0)),
        out_shape=jax.ShapeDtypeStruct((n, d_out), jnp.float32),
        compiler_params=pltpu.CompilerParams(
            dimension_semantics=("parallel", "arbitrary"),
        ),
    )(scal, dist_matrix, h)
    return out
